# baseline (speedup 1.0000x reference)
import jax
import jax.numpy as jnp
from jax.experimental import pallas as pl
from jax.experimental.pallas import tpu as pltpu

_B, _J, _H, _W = 128, 17, 64, 48
_HW = _H * _W
_RB = 768
_GRID = _HW // _RB
_TOPK = 8


def _ohkm_kernel(o_ref, t_ref, w_ref, out_ref, s_ref):
    i = pl.program_id(0)
    d = o_ref[...] - t_ref[...]
    part = jnp.sum(d * d, axis=1)

    @pl.when(i == 0)
    def _():
        s_ref[...] = jnp.zeros((_J, _B), jnp.float32)

    s_ref[...] += part

    @pl.when(i == _GRID - 1)
    def _():
        w = w_ref[...]
        vals = s_ref[...] * (w * w) * (0.5 / _HW)
        row = jax.lax.broadcasted_iota(jnp.int32, vals.shape, 0)
        acc = jnp.zeros((_B,), jnp.float32)
        neg_inf = jnp.float32(-jnp.inf)
        for _ in range(_TOPK):
            m = jnp.max(vals, axis=0)
            acc = acc + m
            is_max = vals == m[None, :]
            first_idx = jnp.min(jnp.where(is_max, row, _J), axis=0)
            vals = jnp.where(row == first_idx[None, :], neg_inf, vals)
        out_ref[0, 0] = jnp.sum(acc) * (1.0 / (_TOPK * _B))


def kernel(outs, targets, target_weights):
    o = jnp.transpose(outs, (1, 2, 3, 0)).reshape(_J, _HW, _B)
    t = jnp.transpose(targets, (1, 2, 3, 0)).reshape(_J, _HW, _B)
    w = jnp.transpose(target_weights, (1, 2, 0)).reshape(_J, _B)
    out = pl.pallas_call(
        _ohkm_kernel,
        grid=(_GRID,),
        in_specs=[
            pl.BlockSpec((_J, _RB, _B), lambda i: (0, i, 0)),
            pl.BlockSpec((_J, _RB, _B), lambda i: (0, i, 0)),
            pl.BlockSpec((_J, _B), lambda i: (0, 0)),
        ],
        out_specs=pl.BlockSpec(
            (1, 1), lambda i: (0, 0), memory_space=pltpu.SMEM
        ),
        out_shape=jax.ShapeDtypeStruct((1, 1), jnp.float32),
        scratch_shapes=[pltpu.VMEM((_J, _B), jnp.float32)],
    )(o, t, w)
    return out.reshape(())


# speedup vs baseline: 1.0452x; 1.0452x over previous
import jax
import jax.numpy as jnp
from jax.experimental import pallas as pl
from jax.experimental.pallas import tpu as pltpu

_B, _J, _H, _W = 128, 17, 64, 48
_HW = _H * _W
_RB = 512
_GRID = _HW // _RB
_TOPK = 8


def _ohkm_kernel(o_ref, t_ref, w_ref, out_ref, s_ref):
    i = pl.program_id(0)
    d = o_ref[...] - t_ref[...]
    part = jnp.sum(d * d, axis=1)

    @pl.when(i == 0)
    def _():
        s_ref[...] = jnp.zeros((_J, _B), jnp.float32)

    s_ref[...] += part

    @pl.when(i == _GRID - 1)
    def _():
        w = w_ref[...]
        vals = s_ref[...] * (w * w) * (0.5 / _HW)
        row = jax.lax.broadcasted_iota(jnp.int32, vals.shape, 0)
        acc = jnp.zeros((_B,), jnp.float32)
        neg_inf = jnp.float32(-jnp.inf)
        for _ in range(_TOPK):
            m = jnp.max(vals, axis=0)
            acc = acc + m
            is_max = vals == m[None, :]
            first_idx = jnp.min(jnp.where(is_max, row, _J), axis=0)
            vals = jnp.where(row == first_idx[None, :], neg_inf, vals)
        out_ref[0, 0] = jnp.sum(acc) * (1.0 / (_TOPK * _B))


def kernel(outs, targets, target_weights):
    o = jnp.transpose(outs, (1, 2, 3, 0)).reshape(_J, _HW, _B)
    t = jnp.transpose(targets, (1, 2, 3, 0)).reshape(_J, _HW, _B)
    w = jnp.transpose(target_weights, (1, 2, 0)).reshape(_J, _B)
    out = pl.pallas_call(
        _ohkm_kernel,
        grid=(_GRID,),
        in_specs=[
            pl.BlockSpec((_J, _RB, _B), lambda i: (0, i, 0)),
            pl.BlockSpec((_J, _RB, _B), lambda i: (0, i, 0)),
            pl.BlockSpec((_J, _B), lambda i: (0, 0)),
        ],
        out_specs=pl.BlockSpec(
            (1, 1), lambda i: (0, 0), memory_space=pltpu.SMEM
        ),
        out_shape=jax.ShapeDtypeStruct((1, 1), jnp.float32),
        scratch_shapes=[pltpu.VMEM((_J, _B), jnp.float32)],
    )(o, t, w)
    return out.reshape(())
